# baseline (speedup 1.0000x reference)
import functools

import jax
import jax.numpy as jnp
from jax import lax
from jax.experimental import pallas as pl
from jax.experimental.pallas import tpu as pltpu
from jax.experimental.pallas import tpu_sc as plsc

NUM_CODES = 8192
DIM = 256
NUM_TOKENS = 8192
TOK_BLOCK = 1024
CODE_CHUNK = 512
_BIG_I32 = 2**30


def _argmin_body(x_ref, fnorm_ref, cbn_ref, cb_ref, idx_ref):
    x = x_ref[0].T
    fn = fnorm_ref[...]
    iota = lax.broadcasted_iota(jnp.int32, (TOK_BLOCK, CODE_CHUNK), 1)
    best = None
    for j in range(NUM_CODES // CODE_CHUNK):
        lo = j * CODE_CHUNK
        cbj = cb_ref[lo:lo + CODE_CHUNK, :]
        mm = lax.dot_general(x, cbj, (((1,), (1,)), ((), ())),
                             preferred_element_type=jnp.float32)
        d = (fn - 2.0 * mm) + cbn_ref[:, lo:lo + CODE_CHUNK]
        m = jnp.min(d, axis=1, keepdims=True)
        li = jnp.min(jnp.where(d == m, iota, _BIG_I32),
                     axis=1, keepdims=True) + lo
        if best is None:
            best, besti = m, li
        else:
            upd = m < best
            besti = jnp.where(upd, li, besti)
            best = jnp.where(upd, m, best)
    idx_ref[...] = besti


_argmin_call = pl.pallas_call(
    _argmin_body,
    grid=(NUM_TOKENS // TOK_BLOCK,),
    in_specs=[
        pl.BlockSpec((1, DIM, TOK_BLOCK), lambda b: (b, 0, 0)),
        pl.BlockSpec((TOK_BLOCK, 1), lambda b: (b, 0)),
        pl.BlockSpec((1, NUM_CODES), lambda b: (0, 0)),
        pl.BlockSpec((NUM_CODES, DIM), lambda b: (0, 0)),
    ],
    out_specs=pl.BlockSpec((TOK_BLOCK, 1), lambda b: (b, 0)),
    out_shape=jax.ShapeDtypeStruct((NUM_TOKENS, 1), jnp.int32),
)


@functools.lru_cache(maxsize=1)
def _make_gather():
    nw = 32
    bpw = NUM_TOKENS // nw
    mesh = plsc.VectorSubcoreMesh(core_axis_name="c", subcore_axis_name="s")

    @functools.partial(
        pl.kernel,
        out_type=jax.ShapeDtypeStruct((NUM_TOKENS, DIM), jnp.float32),
        mesh=mesh,
        scratch_types=[
            pltpu.VMEM((bpw,), jnp.int32),
            pltpu.VMEM((bpw, DIM), jnp.float32),
            pltpu.SemaphoreType.DMA,
        ],
    )
    def gather_k(table_hbm, idx_hbm, out_hbm, idx_v, rows_v, sem):
        wid = lax.axis_index("s") * 2 + lax.axis_index("c")
        base = wid * bpw
        pltpu.sync_copy(idx_hbm.at[pl.ds(base, bpw)], idx_v)
        pltpu.async_copy(table_hbm.at[idx_v], rows_v, sem).wait()
        pltpu.sync_copy(rows_v, out_hbm.at[pl.ds(base, bpw)])

    return gather_k


def _transpose_body(q_ref, out_ref):
    out_ref[0] = q_ref[0].T


_transpose_call = pl.pallas_call(
    _transpose_body,
    grid=(NUM_TOKENS // TOK_BLOCK,),
    in_specs=[pl.BlockSpec((1, TOK_BLOCK, DIM), lambda b: (b, 0, 0))],
    out_specs=pl.BlockSpec((1, DIM, TOK_BLOCK), lambda b: (b, 0, 0)),
    out_shape=jax.ShapeDtypeStruct((NUM_TOKENS // TOK_BLOCK, DIM, TOK_BLOCK),
                                   jnp.float32),
)


def kernel(input, codebook):
    B, C, H, W = input.shape
    x3 = input.reshape(B, C, H * W)
    fnorm = jnp.sum(input ** 2, axis=1).reshape(-1, 1)
    cbnorm = jnp.sum(codebook ** 2, axis=1)[None, :]
    idx = _argmin_call(x3, fnorm, cbnorm, codebook)
    q = _make_gather()(codebook, idx.reshape(-1))
    out = _transpose_call(q.reshape(B, H * W, DIM)).reshape(B, C, H, W)
    return out, out


# speedup vs baseline: 1.0019x; 1.0019x over previous
import functools

import jax
import jax.numpy as jnp
from jax import lax
from jax.experimental import pallas as pl
from jax.experimental.pallas import tpu as pltpu
from jax.experimental.pallas import tpu_sc as plsc

NUM_CODES = 8192
DIM = 256
NUM_TOKENS = 8192
TOK_BLOCK = 1024
CODE_CHUNK = 512
_BIG_I32 = 2**30


def _argmin_body(x_ref, fnorm_ref, cbn_ref, cb_ref, idx_ref):
    x = x_ref[0].T
    fn = fnorm_ref[...]
    iota = lax.broadcasted_iota(jnp.int32, (TOK_BLOCK, CODE_CHUNK), 1)
    nchunks = NUM_CODES // CODE_CHUNK

    def chunk_mm(j):
        lo = j * CODE_CHUNK
        cbj = cb_ref[lo:lo + CODE_CHUNK, :]
        return lax.dot_general(x, cbj, (((1,), (1,)), ((), ())),
                               preferred_element_type=jnp.float32)

    def chunk_reduce(j, mm):
        lo = j * CODE_CHUNK
        d = (fn - 2.0 * mm) + cbn_ref[:, lo:lo + CODE_CHUNK]
        m = jnp.min(d, axis=1, keepdims=True)
        li = jnp.min(jnp.where(d == m, iota, _BIG_I32),
                     axis=1, keepdims=True) + lo
        return m, li

    best = besti = None
    mm = chunk_mm(0)
    for j in range(nchunks):
        mm_next = chunk_mm(j + 1) if j + 1 < nchunks else None
        m, li = chunk_reduce(j, mm)
        if best is None:
            best, besti = m, li
        else:
            upd = m < best
            besti = jnp.where(upd, li, besti)
            best = jnp.where(upd, m, best)
        mm = mm_next
    idx_ref[...] = besti


_argmin_call = pl.pallas_call(
    _argmin_body,
    grid=(NUM_TOKENS // TOK_BLOCK,),
    in_specs=[
        pl.BlockSpec((1, DIM, TOK_BLOCK), lambda b: (b, 0, 0)),
        pl.BlockSpec((TOK_BLOCK, 1), lambda b: (b, 0)),
        pl.BlockSpec((1, NUM_CODES), lambda b: (0, 0)),
        pl.BlockSpec((NUM_CODES, DIM), lambda b: (0, 0)),
    ],
    out_specs=pl.BlockSpec((TOK_BLOCK, 1), lambda b: (b, 0)),
    out_shape=jax.ShapeDtypeStruct((NUM_TOKENS, 1), jnp.int32),
)


@functools.lru_cache(maxsize=1)
def _make_gather():
    nw = 32
    bpw = NUM_TOKENS // nw
    mesh = plsc.VectorSubcoreMesh(core_axis_name="c", subcore_axis_name="s")

    @functools.partial(
        pl.kernel,
        out_type=jax.ShapeDtypeStruct((NUM_TOKENS, DIM), jnp.float32),
        mesh=mesh,
        scratch_types=[
            pltpu.VMEM((bpw,), jnp.int32),
            pltpu.VMEM((bpw, DIM), jnp.float32),
            pltpu.SemaphoreType.DMA,
        ],
    )
    def gather_k(table_hbm, idx_hbm, out_hbm, idx_v, rows_v, sem):
        wid = lax.axis_index("s") * 2 + lax.axis_index("c")
        base = wid * bpw
        pltpu.sync_copy(idx_hbm.at[pl.ds(base, bpw)], idx_v)
        pltpu.async_copy(table_hbm.at[idx_v], rows_v, sem).wait()
        pltpu.sync_copy(rows_v, out_hbm.at[pl.ds(base, bpw)])

    return gather_k


def _transpose_body(q_ref, out_ref):
    out_ref[0] = q_ref[0].T


_transpose_call = pl.pallas_call(
    _transpose_body,
    grid=(NUM_TOKENS // TOK_BLOCK,),
    in_specs=[pl.BlockSpec((1, TOK_BLOCK, DIM), lambda b: (b, 0, 0))],
    out_specs=pl.BlockSpec((1, DIM, TOK_BLOCK), lambda b: (b, 0, 0)),
    out_shape=jax.ShapeDtypeStruct((NUM_TOKENS // TOK_BLOCK, DIM, TOK_BLOCK),
                                   jnp.float32),
)


def kernel(input, codebook):
    B, C, H, W = input.shape
    x3 = input.reshape(B, C, H * W)
    fnorm = jnp.sum(input ** 2, axis=1).reshape(-1, 1)
    cbnorm = jnp.sum(codebook ** 2, axis=1)[None, :]
    idx = _argmin_call(x3, fnorm, cbnorm, codebook)
    q = _make_gather()(codebook, idx.reshape(-1))
    out = _transpose_call(q.reshape(B, H * W, DIM)).reshape(B, C, H, W)
    return out, out


# speedup vs baseline: 1.1472x; 1.1450x over previous
import functools

import jax
import jax.numpy as jnp
from jax import lax
from jax.experimental import pallas as pl
from jax.experimental.pallas import tpu as pltpu
from jax.experimental.pallas import tpu_sc as plsc

NUM_CODES = 8192
DIM = 256
NUM_TOKENS = 8192
TOK_BLOCK = 1024
CODE_CHUNK = 512
_BIG_I32 = 2**30


def _argmin_body(x_ref, fnorm_ref, cbn_ref, cb_ref, idx_ref):
    x = x_ref[0].T
    fn = fnorm_ref[...]
    fiota = lax.broadcasted_iota(jnp.int32, (TOK_BLOCK, CODE_CHUNK), 1
                                 ).astype(jnp.float32)
    nchunks = NUM_CODES // CODE_CHUNK

    def chunk_mm(j):
        lo = j * CODE_CHUNK
        cbj = cb_ref[lo:lo + CODE_CHUNK, :]
        return lax.dot_general(x, cbj, (((1,), (1,)), ((), ())),
                               preferred_element_type=jnp.float32)

    def chunk_reduce(j, mm):
        lo = j * CODE_CHUNK
        d = (fn - 2.0 * mm) + cbn_ref[:, lo:lo + CODE_CHUNK]
        m = jnp.min(d, axis=1, keepdims=True)
        li_f = jnp.min(jnp.where(d == m, fiota, jnp.float32(1e30)),
                       axis=1, keepdims=True)
        li = li_f.astype(jnp.int32) + lo
        return m, li

    best = besti = None
    mm = chunk_mm(0)
    for j in range(nchunks):
        mm_next = chunk_mm(j + 1) if j + 1 < nchunks else None
        m, li = chunk_reduce(j, mm)
        if best is None:
            best, besti = m, li
        else:
            upd = m < best
            besti = jnp.where(upd, li, besti)
            best = jnp.where(upd, m, best)
        mm = mm_next
    idx_ref[...] = besti


_argmin_call = pl.pallas_call(
    _argmin_body,
    grid=(NUM_TOKENS // TOK_BLOCK,),
    in_specs=[
        pl.BlockSpec((1, DIM, TOK_BLOCK), lambda b: (b, 0, 0)),
        pl.BlockSpec((TOK_BLOCK, 1), lambda b: (b, 0)),
        pl.BlockSpec((1, NUM_CODES), lambda b: (0, 0)),
        pl.BlockSpec((NUM_CODES, DIM), lambda b: (0, 0)),
    ],
    out_specs=pl.BlockSpec((TOK_BLOCK, 1), lambda b: (b, 0)),
    out_shape=jax.ShapeDtypeStruct((NUM_TOKENS, 1), jnp.int32),
)


@functools.lru_cache(maxsize=1)
def _make_gather():
    nw = 32
    bpw = NUM_TOKENS // nw
    mesh = plsc.VectorSubcoreMesh(core_axis_name="c", subcore_axis_name="s")

    @functools.partial(
        pl.kernel,
        out_type=jax.ShapeDtypeStruct((NUM_TOKENS, DIM), jnp.float32),
        mesh=mesh,
        scratch_types=[
            pltpu.VMEM((bpw,), jnp.int32),
            pltpu.VMEM((bpw, DIM), jnp.float32),
            pltpu.SemaphoreType.DMA,
        ],
    )
    def gather_k(table_hbm, idx_hbm, out_hbm, idx_v, rows_v, sem):
        wid = lax.axis_index("s") * 2 + lax.axis_index("c")
        base = wid * bpw
        pltpu.sync_copy(idx_hbm.at[pl.ds(base, bpw)], idx_v)
        pltpu.async_copy(table_hbm.at[idx_v], rows_v, sem).wait()
        pltpu.sync_copy(rows_v, out_hbm.at[pl.ds(base, bpw)])

    return gather_k


def _transpose_body(q_ref, out_ref):
    out_ref[0] = q_ref[0].T


_transpose_call = pl.pallas_call(
    _transpose_body,
    grid=(NUM_TOKENS // TOK_BLOCK,),
    in_specs=[pl.BlockSpec((1, TOK_BLOCK, DIM), lambda b: (b, 0, 0))],
    out_specs=pl.BlockSpec((1, DIM, TOK_BLOCK), lambda b: (b, 0, 0)),
    out_shape=jax.ShapeDtypeStruct((NUM_TOKENS // TOK_BLOCK, DIM, TOK_BLOCK),
                                   jnp.float32),
)


def kernel(input, codebook):
    B, C, H, W = input.shape
    x3 = input.reshape(B, C, H * W)
    fnorm = jnp.sum(input ** 2, axis=1).reshape(-1, 1)
    cbnorm = jnp.sum(codebook ** 2, axis=1)[None, :]
    idx = _argmin_call(x3, fnorm, cbnorm, codebook)
    q = _make_gather()(codebook, idx.reshape(-1))
    out = _transpose_call(q.reshape(B, H * W, DIM)).reshape(B, C, H, W)
    return out, out
